# baseline (speedup 1.0000x reference)
import jax
import jax.numpy as jnp
from jax import lax
from jax.experimental import pallas as pl
from jax.experimental.pallas import tpu as pltpu
from jax.experimental.pallas import tpu_sc as plsc

D_MODEL = 64
SCALE = float(D_MODEL) ** 0.5
NUM_WORKERS = 32
NTC = 128 // NUM_WORKERS
SEQ = 50
NBLK = SEQ * NTC


def _emb_body(xt_hbm, table_hbm, out_hbm, idx2, gbufs, tbufs, sems_g, sems_o):
    w = lax.axis_index("s") * 2 + lax.axis_index("c")
    tc0 = w * NTC

    pltpu.sync_copy(xt_hbm.at[:, pl.ds(tc0 * 128, NTC * 128)], idx2)

    base16 = lax.iota(jnp.int32, 16)

    def gather_desc(blk, j, g):
        return pltpu.make_async_copy(
            table_hbm.at[idx2.at[blk, pl.ds(j * 128, 128)]], gbufs[g],
            sems_g[g])

    def out_descs(blk, tcj, p):
        return [
            pltpu.make_async_copy(
                tbufs[p].at[pl.ds(tr * 8, 8), pl.ds(0, 128)],
                out_hbm.at[blk, tr, tc0 + tcj], sems_o[p])
            for tr in range(8)
        ]

    dvecs = [base16 + 16 * j for j in range(D_MODEL // 16)]

    def transpose_scale(g, p):
        @plsc.parallel_loop(0, 128, unroll=16)
        def tbody(t):
            tvec = jnp.zeros((16,), jnp.int32) + t
            for j in range(D_MODEL // 16):
                vals = gbufs[g][t, pl.ds(j * 16, 16)] * SCALE
                plsc.store_scatter(tbufs[p], [dvecs[j], tvec], vals)

    def step(blk, j, skip_wait_out=False, skip_gather=False):
        p = j % 2
        if not skip_gather:
            if j < 2:
                gather_desc(blk, j + 2, j + 2).start()
            else:
                gather_desc(blk + 1, j - 2, j - 2).start()
        gather_desc(blk, j, j).wait()
        if not skip_wait_out:
            if j < 2:
                for dsc in out_descs(blk - 1, j + 2, p):
                    dsc.wait()
            else:
                for dsc in out_descs(blk, j - 2, p):
                    dsc.wait()
        transpose_scale(j, p)
        for dsc in out_descs(blk, j, p):
            dsc.start()

    gather_desc(0, 0, 0).start()
    gather_desc(0, 1, 1).start()

    step(0, 0, skip_wait_out=True)
    step(0, 1, skip_wait_out=True)
    step(0, 2)
    step(0, 3)

    def sb_body(blk, carry):
        for j in range(NTC):
            step(blk, j)
        return carry

    lax.fori_loop(1, SEQ - 1, sb_body, 0)

    step(SEQ - 1, 0)
    step(SEQ - 1, 1)
    step(SEQ - 1, 2, skip_gather=True)
    step(SEQ - 1, 3, skip_gather=True)

    for dsc in out_descs(SEQ - 1, 2, 0):
        dsc.wait()
    for dsc in out_descs(SEQ - 1, 3, 1):
        dsc.wait()


@jax.jit
def kernel(x, embedding):
    batch, seq = x.shape
    xt = jnp.swapaxes(x, 0, 1).astype(jnp.int32)

    mesh = plsc.VectorSubcoreMesh(core_axis_name="c", subcore_axis_name="s")
    v5 = pl.kernel(
        _emb_body,
        out_type=jax.ShapeDtypeStruct(
            (seq, D_MODEL // 8, batch // 128, 8, 128), jnp.float32),
        mesh=mesh,
        scratch_types=[
            pltpu.VMEM((SEQ, NTC * 128), jnp.int32),
            [pltpu.VMEM((128, D_MODEL), jnp.float32) for _ in range(4)],
            [pltpu.VMEM((D_MODEL, 129), jnp.float32) for _ in range(2)],
            [pltpu.SemaphoreType.DMA for _ in range(4)],
            [pltpu.SemaphoreType.DMA for _ in range(2)],
        ],
        compiler_params=pltpu.CompilerParams(
            use_tc_tiling_on_sc=False, needs_layout_passes=False),
    )(xt, embedding)
    out = jnp.transpose(v5, (2, 4, 0, 1, 3)).reshape(batch, seq, D_MODEL)
    return out


# speedup vs baseline: 1.0533x; 1.0533x over previous
import jax
import jax.numpy as jnp
from jax import lax
from jax.experimental import pallas as pl
from jax.experimental.pallas import tpu as pltpu
from jax.experimental.pallas import tpu_sc as plsc

D_MODEL = 64
SCALE = float(D_MODEL) ** 0.5
NUM_WORKERS = 32
NTC = 128 // NUM_WORKERS
SEQ = 50
NBLK = SEQ * NTC


def _emb_body(xt_hbm, table_hbm, out_hbm, idx2, gbufs, tbufs, sems_g, sems_o):
    w = lax.axis_index("s") * 2 + lax.axis_index("c")
    tc0 = w * NTC

    pltpu.sync_copy(xt_hbm.at[:, pl.ds(tc0 * 128, NTC * 128)], idx2)

    base16 = lax.iota(jnp.int32, 16)

    def gather_desc(blk, j, g):
        return pltpu.make_async_copy(
            table_hbm.at[idx2.at[blk, pl.ds(j * 128, 128)]], gbufs[g],
            sems_g[g])

    def out_descs(blk, tcj, p):
        return [
            pltpu.make_async_copy(
                tbufs[p].at[pl.ds(tr * 8, 8), pl.ds(0, 128)],
                out_hbm.at[blk, tr, tc0 + tcj], sems_o[p])
            for tr in range(8)
        ]

    dvecs = [base16 + 16 * j for j in range(D_MODEL // 16)]

    def transpose_scale(g, p):
        @plsc.parallel_loop(0, 128, unroll=4)
        def tbody(t):
            tvec = jnp.zeros((16,), jnp.int32) + t
            for j in range(D_MODEL // 16):
                vals = gbufs[g][t, pl.ds(j * 16, 16)] * SCALE
                plsc.store_scatter(tbufs[p], [dvecs[j], tvec], vals)

    def step(blk, j, skip_wait_out=False, skip_gather=False):
        p = j % 2
        if not skip_gather:
            if j < 2:
                gather_desc(blk, j + 2, j + 2).start()
            else:
                gather_desc(blk + 1, j - 2, j - 2).start()
        gather_desc(blk, j, j).wait()
        if not skip_wait_out:
            if j < 2:
                for dsc in out_descs(blk - 1, j + 2, p):
                    dsc.wait()
            else:
                for dsc in out_descs(blk, j - 2, p):
                    dsc.wait()
        transpose_scale(j, p)
        for dsc in out_descs(blk, j, p):
            dsc.start()

    gather_desc(0, 0, 0).start()
    gather_desc(0, 1, 1).start()

    step(0, 0, skip_wait_out=True)
    step(0, 1, skip_wait_out=True)
    step(0, 2)
    step(0, 3)

    def sb_body(blk, carry):
        for j in range(NTC):
            step(blk, j)
        return carry

    lax.fori_loop(1, SEQ - 1, sb_body, 0)

    step(SEQ - 1, 0)
    step(SEQ - 1, 1)
    step(SEQ - 1, 2, skip_gather=True)
    step(SEQ - 1, 3, skip_gather=True)

    for dsc in out_descs(SEQ - 1, 2, 0):
        dsc.wait()
    for dsc in out_descs(SEQ - 1, 3, 1):
        dsc.wait()


@jax.jit
def kernel(x, embedding):
    batch, seq = x.shape
    xt = jnp.swapaxes(x, 0, 1).astype(jnp.int32)

    mesh = plsc.VectorSubcoreMesh(core_axis_name="c", subcore_axis_name="s")
    v5 = pl.kernel(
        _emb_body,
        out_type=jax.ShapeDtypeStruct(
            (seq, D_MODEL // 8, batch // 128, 8, 128), jnp.float32),
        mesh=mesh,
        scratch_types=[
            pltpu.VMEM((SEQ, NTC * 128), jnp.int32),
            [pltpu.VMEM((128, D_MODEL), jnp.float32) for _ in range(4)],
            [pltpu.VMEM((D_MODEL, 129), jnp.float32) for _ in range(2)],
            [pltpu.SemaphoreType.DMA for _ in range(4)],
            [pltpu.SemaphoreType.DMA for _ in range(2)],
        ],
        compiler_params=pltpu.CompilerParams(
            use_tc_tiling_on_sc=False, needs_layout_passes=False),
    )(xt, embedding)
    out = jnp.transpose(v5, (2, 4, 0, 1, 3)).reshape(batch, seq, D_MODEL)
    return out


# speedup vs baseline: 1.0550x; 1.0017x over previous
import jax
import jax.numpy as jnp
from jax import lax
from jax.experimental import pallas as pl
from jax.experimental.pallas import tpu as pltpu
from jax.experimental.pallas import tpu_sc as plsc

D_MODEL = 64
SCALE = float(D_MODEL) ** 0.5
NUM_WORKERS = 32
NTC = 128 // NUM_WORKERS
SEQ = 50
NBLK = SEQ * NTC


def _emb_body(xt_hbm, table_hbm, out_hbm, idx2, gbufs, tbufs, sems_g, sems_o):
    w = lax.axis_index("s") * 2 + lax.axis_index("c")
    tc0 = w * NTC

    pltpu.sync_copy(xt_hbm.at[:, pl.ds(tc0 * 128, NTC * 128)], idx2)

    base16 = lax.iota(jnp.int32, 16)

    def gather_desc(blk, j, g):
        return pltpu.make_async_copy(
            table_hbm.at[idx2.at[blk, pl.ds(j * 128, 128)]], gbufs[g],
            sems_g[g])

    def out_descs(blk, tcj, p):
        return [
            pltpu.make_async_copy(
                tbufs[p].at[:, :, pl.ds(0, 128)],
                out_hbm.at[blk, :, tc0 + tcj], sems_o[p])
        ]

    dvecs = [base16 + 16 * j for j in range(D_MODEL // 16)]
    trvecs = [lax.shift_right_logical(d, 3) for d in dvecs]
    rvecs = [d & 7 for d in dvecs]

    def transpose_scale(g, p):
        @plsc.parallel_loop(0, 128, unroll=4)
        def tbody(t):
            tvec = jnp.zeros((16,), jnp.int32) + t
            for j in range(D_MODEL // 16):
                vals = gbufs[g][t, pl.ds(j * 16, 16)] * SCALE
                plsc.store_scatter(
                    tbufs[p], [trvecs[j], rvecs[j], tvec], vals)

    def step(blk, j, skip_wait_out=False, skip_gather=False):
        p = j % 2
        if not skip_gather:
            if j < 2:
                gather_desc(blk, j + 2, j + 2).start()
            else:
                gather_desc(blk + 1, j - 2, j - 2).start()
        gather_desc(blk, j, j).wait()
        if not skip_wait_out:
            if j < 2:
                for dsc in out_descs(blk - 1, j + 2, p):
                    dsc.wait()
            else:
                for dsc in out_descs(blk, j - 2, p):
                    dsc.wait()
        transpose_scale(j, p)
        for dsc in out_descs(blk, j, p):
            dsc.start()

    gather_desc(0, 0, 0).start()
    gather_desc(0, 1, 1).start()

    step(0, 0, skip_wait_out=True)
    step(0, 1, skip_wait_out=True)
    step(0, 2)
    step(0, 3)

    def sb_body(blk, carry):
        for j in range(NTC):
            step(blk, j)
        return carry

    lax.fori_loop(1, SEQ - 1, sb_body, 0)

    step(SEQ - 1, 0)
    step(SEQ - 1, 1)
    step(SEQ - 1, 2, skip_gather=True)
    step(SEQ - 1, 3, skip_gather=True)

    for dsc in out_descs(SEQ - 1, 2, 0):
        dsc.wait()
    for dsc in out_descs(SEQ - 1, 3, 1):
        dsc.wait()


@jax.jit
def kernel(x, embedding):
    batch, seq = x.shape
    xt = jnp.swapaxes(x, 0, 1).astype(jnp.int32)

    mesh = plsc.VectorSubcoreMesh(core_axis_name="c", subcore_axis_name="s")
    v5 = pl.kernel(
        _emb_body,
        out_type=jax.ShapeDtypeStruct(
            (seq, D_MODEL // 8, batch // 128, 8, 128), jnp.float32),
        mesh=mesh,
        scratch_types=[
            pltpu.VMEM((SEQ, NTC * 128), jnp.int32),
            [pltpu.VMEM((128, D_MODEL), jnp.float32) for _ in range(4)],
            [pltpu.VMEM((8, 8, 129), jnp.float32) for _ in range(2)],
            [pltpu.SemaphoreType.DMA for _ in range(4)],
            [pltpu.SemaphoreType.DMA for _ in range(2)],
        ],
        compiler_params=pltpu.CompilerParams(
            use_tc_tiling_on_sc=False, needs_layout_passes=False),
    )(xt, embedding)
    out = jnp.transpose(v5, (2, 4, 0, 1, 3)).reshape(batch, seq, D_MODEL)
    return out
